# baseline (speedup 1.0000x reference)
import functools

import jax
import jax.numpy as jnp
from jax.experimental import pallas as pl
from jax.experimental.pallas import tpu as pltpu

TEMP = 0.5


def _block(n, cap):
    for b in range(min(n, cap), 7, -1):
        if n % b == 0 and b % 8 == 0:
            return b
    return n


def _mlp_kernel(feat_ref, w1_ref, b1_ref, w2_ref, b2_ref, g1w_ref, out_ref):
    f = jnp.maximum(
        jnp.dot(feat_ref[...], w1_ref[...], preferred_element_type=jnp.float32)
        + b1_ref[...], 0.0)
    f = jnp.dot(f, w2_ref[...], preferred_element_type=jnp.float32) + b2_ref[...]
    out_ref[...] = jnp.dot(f, g1w_ref[...], preferred_element_type=jnp.float32)


def _adj_mid_kernel(adj_ref, t_ref, g1b_ref, g2w_ref, out_ref):
    acc = jnp.dot(adj_ref[...].astype(jnp.bfloat16),
                  t_ref[...].astype(jnp.bfloat16),
                  preferred_element_type=jnp.float32)
    h = jnp.maximum(acc + g1b_ref[...], 0.0)
    out_ref[...] = jnp.dot(h, g2w_ref[...], preferred_element_type=jnp.float32)


def _adj_proj_kernel(adj_ref, t_ref, g2b_ref, pw1_ref, pb1_ref, pw2_ref,
                     pb2_ref, out_ref):
    acc = jnp.dot(adj_ref[...].astype(jnp.bfloat16),
                  t_ref[...].astype(jnp.bfloat16),
                  preferred_element_type=jnp.float32)
    h = acc + g2b_ref[...]
    u = jnp.dot(h, pw1_ref[...], preferred_element_type=jnp.float32) + pb1_ref[...]
    e = jnp.where(u > 0.0, u, jnp.exp(jnp.minimum(u, 0.0)) - 1.0)
    z = jnp.dot(e, pw2_ref[...], preferred_element_type=jnp.float32) + pb2_ref[...]
    n = jnp.sqrt(jnp.sum(z * z, axis=1, keepdims=True))
    out_ref[...] = z / jnp.maximum(n, 1e-12)


def _loss_kernel(z1i_ref, z2i_ref, z1j_ref, z2j_ref, out_ref,
                 r11, r12, r21, r22, d11, d12, d22, *, ni, nj, bi, bj, n):
    i = pl.program_id(0)
    j = pl.program_id(1)

    @pl.when((i == 0) & (j == 0))
    def _init():
        for s in (r11, r12, r21, r22, d11, d12, d22):
            s[...] = jnp.zeros_like(s)

    z1i = z1i_ref[...].astype(jnp.bfloat16)
    z2i = z2i_ref[...].astype(jnp.bfloat16)
    z1j = z1j_ref[...].astype(jnp.bfloat16)
    z2j = z2j_ref[...].astype(jnp.bfloat16)
    inv_t = 1.0 / TEMP
    dn = (((1,), (1,)), ((), ()))

    e11 = jnp.exp(jax.lax.dot_general(
        z1i, z1j, dn, preferred_element_type=jnp.float32) * inv_t)
    r11[j] = r11[j] + jnp.sum(e11, axis=0, keepdims=True)
    e22 = jnp.exp(jax.lax.dot_general(
        z2i, z2j, dn, preferred_element_type=jnp.float32) * inv_t)
    r22[j] = r22[j] + jnp.sum(e22, axis=0, keepdims=True)
    e12 = jnp.exp(jax.lax.dot_general(
        z1i, z2j, dn, preferred_element_type=jnp.float32) * inv_t)
    r21[j] = r21[j] + jnp.sum(e12, axis=0, keepdims=True)
    e21 = jnp.exp(jax.lax.dot_general(
        z2i, z1j, dn, preferred_element_type=jnp.float32) * inv_t)
    r12[j] = r12[j] + jnp.sum(e21, axis=0, keepdims=True)

    @pl.when(i == j)
    def _diag():
        mask = (jax.lax.broadcasted_iota(jnp.int32, (bi, bj), 0)
                == jax.lax.broadcasted_iota(jnp.int32, (bi, bj), 1))
        zero = jnp.zeros((), jnp.float32)
        d11[j] = jnp.sum(jnp.where(mask, e11, zero), axis=0, keepdims=True)
        d22[j] = jnp.sum(jnp.where(mask, e22, zero), axis=0, keepdims=True)
        d12[j] = jnp.sum(jnp.where(mask, e12, zero), axis=0, keepdims=True)

    @pl.when((i == ni - 1) & (j == nj - 1))
    def _finish():
        x1 = r11[...] + r12[...] - d11[...]
        x2 = r22[...] + r21[...] - d22[...]
        ld = jnp.log(d12[...])
        l1 = jnp.log(x1) - ld
        l2 = jnp.log(x2) - ld
        out_ref[...] = (jnp.sum((l1 + l2) * 0.5) / n).reshape(1, 1)


def kernel(adj1, adj2, feat1, feat2, W1, b1, W2, b2, g1W, g1b, g2W, g2b,
           pW1, pb1, pW2, pb2):
    n = adj1.shape[0]
    in_dim = feat1.shape[1]
    hid = g1W.shape[1]
    act = g2W.shape[1]

    b1r = b1.reshape(1, -1)
    b2r = b2.reshape(1, -1)
    g1br = g1b.reshape(1, -1)
    g2br = g2b.reshape(1, -1)
    pb1r = pb1.reshape(1, -1)
    pb2r = pb2.reshape(1, -1)

    br_mlp = _block(n, 2000)
    whole = lambda shape: pl.BlockSpec(shape, lambda i: (0, 0))
    mlp_call = pl.pallas_call(
        _mlp_kernel,
        grid=(n // br_mlp,),
        in_specs=[
            pl.BlockSpec((br_mlp, in_dim), lambda i: (i, 0)),
            whole(W1.shape), whole(b1r.shape), whole(W2.shape),
            whole(b2r.shape), whole(g1W.shape),
        ],
        out_specs=pl.BlockSpec((br_mlp, hid), lambda i: (i, 0)),
        out_shape=jax.ShapeDtypeStruct((n, hid), jnp.float32),
    )
    t1a = mlp_call(feat1, W1, b1r, W2, b2r, g1W)
    t1b = mlp_call(feat2, W1, b1r, W2, b2r, g1W)

    br = _block(n, 200)
    mid_call = pl.pallas_call(
        _adj_mid_kernel,
        grid=(n // br,),
        in_specs=[
            pl.BlockSpec((br, n), lambda i: (i, 0)),
            whole((n, hid)), whole(g1br.shape), whole(g2W.shape),
        ],
        out_specs=pl.BlockSpec((br, act), lambda i: (i, 0)),
        out_shape=jax.ShapeDtypeStruct((n, act), jnp.float32),
    )
    t2a = mid_call(adj1, t1a, g1br, g2W)
    t2b = mid_call(adj2, t1b, g1br, g2W)

    proj_call = pl.pallas_call(
        _adj_proj_kernel,
        grid=(n // br,),
        in_specs=[
            pl.BlockSpec((br, n), lambda i: (i, 0)),
            whole((n, act)), whole(g2br.shape), whole(pW1.shape),
            whole(pb1r.shape), whole(pW2.shape), whole(pb2r.shape),
        ],
        out_specs=pl.BlockSpec((br, act), lambda i: (i, 0)),
        out_shape=jax.ShapeDtypeStruct((n, act), jnp.float32),
    )
    z1 = proj_call(adj1, t2a, g2br, pW1, pb1r, pW2, pb2r)
    z2 = proj_call(adj2, t2b, g2br, pW1, pb1r, pW2, pb2r)

    bi = _block(n, 1000)
    bj = bi
    ni = n // bi
    nj = n // bj
    loss_call = pl.pallas_call(
        functools.partial(_loss_kernel, ni=ni, nj=nj, bi=bi, bj=bj, n=float(n)),
        grid=(ni, nj),
        in_specs=[
            pl.BlockSpec((bi, act), lambda i, j: (i, 0)),
            pl.BlockSpec((bi, act), lambda i, j: (i, 0)),
            pl.BlockSpec((bj, act), lambda i, j: (j, 0)),
            pl.BlockSpec((bj, act), lambda i, j: (j, 0)),
        ],
        out_specs=pl.BlockSpec((1, 1), lambda i, j: (0, 0)),
        out_shape=jax.ShapeDtypeStruct((1, 1), jnp.float32),
        scratch_shapes=[pltpu.VMEM((nj, 1, bj), jnp.float32)
                        for _ in range(7)],
    )
    loss = loss_call(z1, z2, z1, z2)
    return loss[0, 0]


# speedup vs baseline: 1.0291x; 1.0291x over previous
import functools

import jax
import jax.numpy as jnp
from jax.experimental import pallas as pl
from jax.experimental.pallas import tpu as pltpu

TEMP = 0.5


def _block(n, cap):
    for b in range(min(n, cap), 7, -1):
        if n % b == 0 and b % 8 == 0:
            return b
    return n


def _mlp_kernel(feat_ref, w1_ref, b1_ref, w2_ref, b2_ref, g1w_ref, out_ref):
    f = jnp.maximum(
        jnp.dot(feat_ref[...], w1_ref[...], preferred_element_type=jnp.float32)
        + b1_ref[...], 0.0)
    f = jnp.dot(f, w2_ref[...], preferred_element_type=jnp.float32) + b2_ref[...]
    out_ref[...] = jnp.dot(f, g1w_ref[...], preferred_element_type=jnp.float32)


def _adj_mid_kernel(adj_ref, t_ref, g1b_ref, g2w_ref, out_ref):
    acc = jnp.dot(adj_ref[...].astype(jnp.bfloat16),
                  t_ref[...].astype(jnp.bfloat16),
                  preferred_element_type=jnp.float32)
    h = jnp.maximum(acc + g1b_ref[...], 0.0)
    out_ref[...] = jnp.dot(h, g2w_ref[...], preferred_element_type=jnp.float32)


def _adj_proj_kernel(adj_ref, t_ref, g2b_ref, pw1_ref, pb1_ref, pw2_ref,
                     pb2_ref, out_ref):
    acc = jnp.dot(adj_ref[...].astype(jnp.bfloat16),
                  t_ref[...].astype(jnp.bfloat16),
                  preferred_element_type=jnp.float32)
    h = acc + g2b_ref[...]
    u = jnp.dot(h, pw1_ref[...], preferred_element_type=jnp.float32) + pb1_ref[...]
    e = jnp.where(u > 0.0, u, jnp.exp(jnp.minimum(u, 0.0)) - 1.0)
    z = jnp.dot(e, pw2_ref[...], preferred_element_type=jnp.float32) + pb2_ref[...]
    n = jnp.sqrt(jnp.sum(z * z, axis=1, keepdims=True))
    out_ref[...] = z / jnp.maximum(n, 1e-12)


def _loss_kernel(z1i_ref, z2i_ref, z1j_ref, z2j_ref, out_ref,
                 r11, r12, r21, r22, d11, d12, d22, *, ni, nj, bi, bj, n):
    i = pl.program_id(0)
    j = pl.program_id(1)

    @pl.when((i == 0) & (j == 0))
    def _init():
        for s in (r11, r12, r21, r22, d11, d12, d22):
            s[...] = jnp.zeros_like(s)

    z1i = z1i_ref[...].astype(jnp.bfloat16)
    z2i = z2i_ref[...].astype(jnp.bfloat16)
    z1j = z1j_ref[...].astype(jnp.bfloat16)
    z2j = z2j_ref[...].astype(jnp.bfloat16)
    inv_t = 1.0 / TEMP
    dn = (((1,), (1,)), ((), ()))

    e11 = jnp.exp(jax.lax.dot_general(
        z1i, z1j, dn, preferred_element_type=jnp.float32) * inv_t)
    r11[j] = r11[j] + jnp.sum(e11, axis=0, keepdims=True)
    e22 = jnp.exp(jax.lax.dot_general(
        z2i, z2j, dn, preferred_element_type=jnp.float32) * inv_t)
    r22[j] = r22[j] + jnp.sum(e22, axis=0, keepdims=True)
    e12 = jnp.exp(jax.lax.dot_general(
        z1i, z2j, dn, preferred_element_type=jnp.float32) * inv_t)
    r21[j] = r21[j] + jnp.sum(e12, axis=0, keepdims=True)
    e21 = jnp.exp(jax.lax.dot_general(
        z2i, z1j, dn, preferred_element_type=jnp.float32) * inv_t)
    r12[j] = r12[j] + jnp.sum(e21, axis=0, keepdims=True)

    @pl.when(i == j)
    def _diag():
        mask = (jax.lax.broadcasted_iota(jnp.int32, (bi, bj), 0)
                == jax.lax.broadcasted_iota(jnp.int32, (bi, bj), 1))
        zero = jnp.zeros((), jnp.float32)
        d11[j] = jnp.sum(jnp.where(mask, e11, zero), axis=0, keepdims=True)
        d22[j] = jnp.sum(jnp.where(mask, e22, zero), axis=0, keepdims=True)
        d12[j] = jnp.sum(jnp.where(mask, e12, zero), axis=0, keepdims=True)

    @pl.when((i == ni - 1) & (j == nj - 1))
    def _finish():
        x1 = r11[...] + r12[...] - d11[...]
        x2 = r22[...] + r21[...] - d22[...]
        ld = jnp.log(d12[...])
        l1 = jnp.log(x1) - ld
        l2 = jnp.log(x2) - ld
        out_ref[...] = (jnp.sum((l1 + l2) * 0.5) / n).reshape(1, 1)


def kernel(adj1, adj2, feat1, feat2, W1, b1, W2, b2, g1W, g1b, g2W, g2b,
           pW1, pb1, pW2, pb2):
    n = adj1.shape[0]
    in_dim = feat1.shape[1]
    hid = g1W.shape[1]
    act = g2W.shape[1]

    b1r = b1.reshape(1, -1)
    b2r = b2.reshape(1, -1)
    g1br = g1b.reshape(1, -1)
    g2br = g2b.reshape(1, -1)
    pb1r = pb1.reshape(1, -1)
    pb2r = pb2.reshape(1, -1)

    br_mlp = _block(n, 2000)
    whole = lambda shape: pl.BlockSpec(shape, lambda i: (0, 0))
    mlp_call = pl.pallas_call(
        _mlp_kernel,
        grid=(n // br_mlp,),
        in_specs=[
            pl.BlockSpec((br_mlp, in_dim), lambda i: (i, 0)),
            whole(W1.shape), whole(b1r.shape), whole(W2.shape),
            whole(b2r.shape), whole(g1W.shape),
        ],
        out_specs=pl.BlockSpec((br_mlp, hid), lambda i: (i, 0)),
        out_shape=jax.ShapeDtypeStruct((n, hid), jnp.float32),
    )
    t1a = mlp_call(feat1, W1, b1r, W2, b2r, g1W)
    t1b = mlp_call(feat2, W1, b1r, W2, b2r, g1W)

    br = _block(n, 400)
    mid_call = pl.pallas_call(
        _adj_mid_kernel,
        grid=(n // br,),
        in_specs=[
            pl.BlockSpec((br, n), lambda i: (i, 0)),
            whole((n, hid)), whole(g1br.shape), whole(g2W.shape),
        ],
        out_specs=pl.BlockSpec((br, act), lambda i: (i, 0)),
        out_shape=jax.ShapeDtypeStruct((n, act), jnp.float32),
    )
    t2a = mid_call(adj1, t1a, g1br, g2W)
    t2b = mid_call(adj2, t1b, g1br, g2W)

    proj_call = pl.pallas_call(
        _adj_proj_kernel,
        grid=(n // br,),
        in_specs=[
            pl.BlockSpec((br, n), lambda i: (i, 0)),
            whole((n, act)), whole(g2br.shape), whole(pW1.shape),
            whole(pb1r.shape), whole(pW2.shape), whole(pb2r.shape),
        ],
        out_specs=pl.BlockSpec((br, act), lambda i: (i, 0)),
        out_shape=jax.ShapeDtypeStruct((n, act), jnp.float32),
    )
    z1 = proj_call(adj1, t2a, g2br, pW1, pb1r, pW2, pb2r)
    z2 = proj_call(adj2, t2b, g2br, pW1, pb1r, pW2, pb2r)

    bi = _block(n, 1000)
    bj = bi
    ni = n // bi
    nj = n // bj
    loss_call = pl.pallas_call(
        functools.partial(_loss_kernel, ni=ni, nj=nj, bi=bi, bj=bj, n=float(n)),
        grid=(ni, nj),
        in_specs=[
            pl.BlockSpec((bi, act), lambda i, j: (i, 0)),
            pl.BlockSpec((bi, act), lambda i, j: (i, 0)),
            pl.BlockSpec((bj, act), lambda i, j: (j, 0)),
            pl.BlockSpec((bj, act), lambda i, j: (j, 0)),
        ],
        out_specs=pl.BlockSpec((1, 1), lambda i, j: (0, 0)),
        out_shape=jax.ShapeDtypeStruct((1, 1), jnp.float32),
        scratch_shapes=[pltpu.VMEM((nj, 1, bj), jnp.float32)
                        for _ in range(7)],
    )
    loss = loss_call(z1, z2, z1, z2)
    return loss[0, 0]
